# Initial kernel scaffold; baseline (speedup 1.0000x reference)
import jax
import jax.numpy as jnp
from jax.experimental import pallas as pl


def kernel(mu, sigma, indices_tuple):
    raise NotImplementedError("write your pallas kernel here")


# speedup vs baseline: 1.5745x; 1.5745x over previous
import functools

import jax
import jax.numpy as jnp
from jax import lax
from jax.experimental import pallas as pl
from jax.experimental.pallas import tpu as pltpu
from jax.experimental.pallas import tpu_sc as plsc

N = 16384
D = 128
P = 16384

NC = 2
NS = 16
L = 16
NW = NC * NS
PPW = P // NW
C = 64
NCH = PPW // C
R = 2 * C

_LN2 = 0.6931471805599453
_INV_PD = 1.0 / (P * D)


def _log_and_recip(s):
  bits = lax.bitcast_convert_type(s, jnp.int32)
  m = lax.bitcast_convert_type(
      (bits & jnp.int32(0x007FFFFF)) | jnp.int32(0x3F800000), jnp.float32)
  ef = (lax.shift_right_logical(bits, 23) - 127).astype(jnp.float32)
  scale = lax.bitcast_convert_type(
      jnp.int32(0x7F000000) - (bits & jnp.int32(0x7F800000)), jnp.float32)
  q = 1.0 / (m * m + m)
  inv_s = (m + 1.0) * q * scale
  r = (m - 1.0) * (m * q)
  r2 = r * r
  poly = 2.0 + r2 * (0.6666666666666667 + r2 * (0.4 + r2 * 0.2857142857142857))
  log_s = ef * _LN2 + r * poly
  return log_s, inv_s


def _sc_partials(mu, sigma, idx):
  mesh = plsc.VectorSubcoreMesh(core_axis_name="c", subcore_axis_name="s")

  @functools.partial(
      pl.kernel,
      out_type=jax.ShapeDtypeStruct((NW * 4, L), jnp.float32),
      mesh=mesh,
      scratch_types=[
          pltpu.VMEM((NCH, R), jnp.int32),
          pltpu.VMEM((2, R, D), jnp.float32),
          pltpu.VMEM((2, R, D), jnp.float32),
          pltpu.VMEM((4, L), jnp.float32),
          pltpu.SemaphoreType.DMA,
          pltpu.SemaphoreType.DMA,
          pltpu.SemaphoreType.DMA,
          pltpu.SemaphoreType.DMA,
      ],
  )
  def sc_kernel(mu_hbm, sg_hbm, idx_hbm, out_hbm,
                idx_v, mub, sgb, outv, m0, m1, s0, s1):
    w = lax.axis_index("s") * NC + lax.axis_index("c")
    pltpu.sync_copy(idx_hbm.at[w], idx_v)
    msems = (m0, m1)
    ssems = (s0, s1)

    def fire(c):
      b = c % 2
      mcp = pltpu.async_copy(mu_hbm.at[idx_v.at[c]], mub.at[b], msems[b])
      scp = pltpu.async_copy(sg_hbm.at[idx_v.at[c]], sgb.at[b], ssems[b])
      return mcp, scp

    def chunk_body(b, accs):
      def body(i, accs):
        a_sq, a_inv, a_log = accs
        for j in range(D // L):
          sl = pl.ds(j * L, L)
          d = mub[b, i, sl] - mub[b, C + i, sl]
          a_sq = a_sq + d * d
          sa = sgb[b, i, sl]
          sp = sgb[b, C + i, sl]
          s = sa * sa + sp * sp
          log_s, inv_s = _log_and_recip(s)
          a_inv = a_inv + inv_s
          a_log = a_log + log_s
        return (a_sq, a_inv, a_log)
      return lax.fori_loop(0, C, body, accs)

    cps = fire(0)
    zero = jnp.zeros((L,), jnp.float32)
    accs = (zero, zero, zero)
    for c in range(NCH):
      mcp, scp = cps
      mcp.wait()
      scp.wait()
      if c + 1 < NCH:
        cps = fire(c + 1)
      accs = chunk_body(c % 2, accs)

    outv[0, :] = accs[0]
    outv[1, :] = accs[1]
    outv[2, :] = accs[2]
    outv[3, :] = zero
    pltpu.sync_copy(outv, out_hbm.at[pl.ds(w * 4, 4)])

  return sc_kernel(mu, sigma, idx)


def _tc_finish(partials):
  def body(p_ref, o_ref):
    x = p_ref[...]
    rid = lax.broadcasted_iota(jnp.int32, (NW * 4, L), 0) % 4
    ssq = jnp.sum(jnp.where(rid == 0, x, 0.0))
    sinv = jnp.sum(jnp.where(rid == 1, x, 0.0))
    slog = jnp.sum(jnp.where(rid == 2, x, 0.0))
    res = jnp.sqrt(ssq) * (sinv * _INV_PD) + slog * _INV_PD
    o_ref[...] = jnp.reshape(res, (1, 1))

  return pl.pallas_call(
      body,
      out_shape=jax.ShapeDtypeStruct((1, 1), jnp.float32),
  )(partials)


def kernel(mu, sigma, indices_tuple):
  a = indices_tuple[0]
  p = indices_tuple[1]
  idx = jnp.concatenate(
      [a.reshape(NW, NCH, 1, C), p.reshape(NW, NCH, 1, C)], axis=2)
  idx = idx.reshape(NW, NCH, R)
  partials = _sc_partials(mu, sigma, idx)
  return _tc_finish(partials)[0, 0]


# speedup vs baseline: 1.8445x; 1.1715x over previous
import functools

import jax
import jax.numpy as jnp
from jax import lax
from jax.experimental import pallas as pl
from jax.experimental.pallas import tpu as pltpu
from jax.experimental.pallas import tpu_sc as plsc

N = 16384
D = 128
P = 16384

NC = 2
NS = 16
L = 16
NW = NC * NS
PPW = P // NW
C = 64
NCH = PPW // C
R = 2 * C

_LN2 = 0.6931471805599453
_INV_PD = 1.0 / (P * D)
_EPL = PPW * D // L


def _sc_partials(mu, sigma, indices_tuple):
  mesh = plsc.VectorSubcoreMesh(core_axis_name="c", subcore_axis_name="s")

  @functools.partial(
      pl.kernel,
      out_type=jax.ShapeDtypeStruct((NW * 4, L), jnp.float32),
      mesh=mesh,
      scratch_types=[
          pltpu.VMEM((2, PPW), jnp.int32),
          pltpu.VMEM((2, R, D), jnp.float32),
          pltpu.VMEM((2, R, D), jnp.float32),
          pltpu.VMEM((4, L), jnp.float32),
          pltpu.SemaphoreType.DMA,
          pltpu.SemaphoreType.DMA,
          pltpu.SemaphoreType.DMA,
          pltpu.SemaphoreType.DMA,
      ],
  )
  def sc_kernel(mu_hbm, sg_hbm, idx_hbm, out_hbm,
                idx_v, mub, sgb, outv, m0, m1, s0, s1):
    w = lax.axis_index("s") * NC + lax.axis_index("c")
    pltpu.sync_copy(idx_hbm.at[0, pl.ds(w * PPW, PPW)], idx_v.at[0])
    pltpu.sync_copy(idx_hbm.at[1, pl.ds(w * PPW, PPW)], idx_v.at[1])
    msems = (m0, m1)
    ssems = (s0, s1)

    def fire(c):
      b = c % 2
      sl = pl.ds(c * C, C)
      cps = (
          pltpu.async_copy(mu_hbm.at[idx_v.at[0, sl]],
                           mub.at[b, pl.ds(0, C)], msems[b]),
          pltpu.async_copy(mu_hbm.at[idx_v.at[1, sl]],
                           mub.at[b, pl.ds(C, C)], msems[b]),
          pltpu.async_copy(sg_hbm.at[idx_v.at[0, sl]],
                           sgb.at[b, pl.ds(0, C)], ssems[b]),
          pltpu.async_copy(sg_hbm.at[idx_v.at[1, sl]],
                           sgb.at[b, pl.ds(C, C)], ssems[b]),
      )
      return cps

    def chunk_body(b, accs):
      def body(i, accs):
        a_sq, a_inv, a_lm, a_e = accs
        for j in range(D // L):
          sl = pl.ds(j * L, L)
          d = mub[b, i, sl] - mub[b, C + i, sl]
          a_sq = a_sq + d * d
          sa = sgb[b, i, sl]
          sp = sgb[b, C + i, sl]
          s = sa * sa + sp * sp
          bits = lax.bitcast_convert_type(s, jnp.int32)
          a_e = a_e + lax.shift_right_logical(bits, 23)
          m = lax.bitcast_convert_type(
              (bits & jnp.int32(0x007FFFFF)) | jnp.int32(0x3F800000),
              jnp.float32)
          a_inv = a_inv + 1.0 / s
          r = (m - 1.0) / (m + 1.0)
          r2 = r * r
          a_lm = a_lm + r * (2.0 + r2 * (0.6666666666666667 + r2 * 0.4))
        return (a_sq, a_inv, a_lm, a_e)
      return lax.fori_loop(0, C, body, accs)

    cps = fire(0)
    zero = jnp.zeros((L,), jnp.float32)
    accs = (zero, zero, zero, jnp.zeros((L,), jnp.int32))
    for c in range(NCH):
      for cp in cps:
        cp.wait()
      if c + 1 < NCH:
        cps = fire(c + 1)
      accs = chunk_body(c % 2, accs)

    outv[0, :] = accs[0]
    outv[1, :] = accs[1]
    outv[2, :] = accs[2] + (accs[3].astype(jnp.float32) - 127.0 * _EPL) * _LN2
    outv[3, :] = zero
    pltpu.sync_copy(outv, out_hbm.at[pl.ds(w * 4, 4)])

  return sc_kernel(mu, sigma, indices_tuple)


def _tc_finish(partials):
  def body(p_ref, o_ref):
    x = p_ref[...]
    rid = lax.broadcasted_iota(jnp.int32, (NW * 4, L), 0) % 4
    ssq = jnp.sum(jnp.where(rid == 0, x, 0.0))
    sinv = jnp.sum(jnp.where(rid == 1, x, 0.0))
    slog = jnp.sum(jnp.where(rid == 2, x, 0.0))
    res = jnp.sqrt(ssq) * (sinv * _INV_PD) + slog * _INV_PD
    o_ref[...] = jnp.reshape(res, (1, 1))

  return pl.pallas_call(
      body,
      out_shape=jax.ShapeDtypeStruct((1, 1), jnp.float32),
  )(partials)


def kernel(mu, sigma, indices_tuple):
  partials = _sc_partials(mu, sigma, indices_tuple)
  return _tc_finish(partials)[0, 0]


# speedup vs baseline: 1.9068x; 1.0338x over previous
import functools

import jax
import jax.numpy as jnp
from jax import lax
from jax.experimental import pallas as pl
from jax.experimental.pallas import tpu as pltpu
from jax.experimental.pallas import tpu_sc as plsc

N = 16384
D = 128
P = 16384

NC = 2
NS = 16
L = 16
NW = NC * NS
PPW = P // NW
C = 64
NCH = PPW // C
R = 2 * C

_LN2 = 0.6931471805599453
_INV_PD = 1.0 / (P * D)
_EPL = PPW * D // L


def _sc_partials(mu, sigma, indices_tuple):
  mesh = plsc.VectorSubcoreMesh(core_axis_name="c", subcore_axis_name="s")

  @functools.partial(
      pl.kernel,
      out_type=jax.ShapeDtypeStruct((NW * 4, L), jnp.float32),
      mesh=mesh,
      scratch_types=[
          pltpu.VMEM((2, PPW), jnp.int32),
          pltpu.VMEM((2, R, D), jnp.float32),
          pltpu.VMEM((2, R, D), jnp.float32),
          pltpu.VMEM((4, L), jnp.float32),
          pltpu.SemaphoreType.DMA,
          pltpu.SemaphoreType.DMA,
          pltpu.SemaphoreType.DMA,
          pltpu.SemaphoreType.DMA,
      ],
  )
  def sc_kernel(mu_hbm, sg_hbm, idx_hbm, out_hbm,
                idx_v, mub, sgb, outv, m0, m1, s0, s1):
    w = lax.axis_index("s") * NC + lax.axis_index("c")
    pltpu.sync_copy(idx_hbm.at[0, pl.ds(w * PPW, PPW)], idx_v.at[0])
    pltpu.sync_copy(idx_hbm.at[1, pl.ds(w * PPW, PPW)], idx_v.at[1])
    msems = (m0, m1)
    ssems = (s0, s1)

    def fire(c):
      b = c % 2
      sl = pl.ds(c * C, C)
      cps = (
          pltpu.async_copy(mu_hbm.at[idx_v.at[0, sl]],
                           mub.at[b, pl.ds(0, C)], msems[b]),
          pltpu.async_copy(mu_hbm.at[idx_v.at[1, sl]],
                           mub.at[b, pl.ds(C, C)], msems[b]),
          pltpu.async_copy(sg_hbm.at[idx_v.at[0, sl]],
                           sgb.at[b, pl.ds(0, C)], ssems[b]),
          pltpu.async_copy(sg_hbm.at[idx_v.at[1, sl]],
                           sgb.at[b, pl.ds(C, C)], ssems[b]),
      )
      return cps

    def chunk_body(b, accs):
      def body(i, accs):
        accs = list(accs)
        for j in range(D // L):
          a_sq, a_inv, a_lm, a_e = accs[4 * (j % 2):4 * (j % 2) + 4]
          sl = pl.ds(j * L, L)
          d = mub[b, i, sl] - mub[b, C + i, sl]
          a_sq = a_sq + d * d
          sa = sgb[b, i, sl]
          sp = sgb[b, C + i, sl]
          s = sa * sa + sp * sp
          bits = lax.bitcast_convert_type(s, jnp.int32)
          a_e = a_e + lax.shift_right_logical(bits, 23)
          m = lax.bitcast_convert_type(
              (bits & jnp.int32(0x007FFFFF)) | jnp.int32(0x3F800000),
              jnp.float32)
          a_inv = a_inv + 1.0 / s
          r = (m - 1.0) / (m + 1.0)
          a_lm = a_lm + r * (2.0 + (r * r) * 0.6666666666666667)
          accs[4 * (j % 2):4 * (j % 2) + 4] = [a_sq, a_inv, a_lm, a_e]
        return tuple(accs)
      return plsc.parallel_loop(0, C, unroll=2, carry=accs)(body)

    cps = fire(0)
    zero = jnp.zeros((L,), jnp.float32)
    izero = jnp.zeros((L,), jnp.int32)
    accs = (zero, zero, zero, izero, zero, zero, zero, izero)
    for c in range(NCH):
      for cp in cps:
        cp.wait()
      if c + 1 < NCH:
        cps = fire(c + 1)
      accs = chunk_body(c % 2, accs)

    a_e = accs[3] + accs[7]
    outv[0, :] = accs[0] + accs[4]
    outv[1, :] = accs[1] + accs[5]
    outv[2, :] = (accs[2] + accs[6]
                  + (a_e.astype(jnp.float32) - 127.0 * _EPL) * _LN2)
    outv[3, :] = zero
    pltpu.sync_copy(outv, out_hbm.at[pl.ds(w * 4, 4)])

  return sc_kernel(mu, sigma, indices_tuple)


def _tc_finish(partials):
  def body(p_ref, o_ref):
    x = p_ref[...]
    rid = lax.broadcasted_iota(jnp.int32, (NW * 4, L), 0) % 4
    ssq = jnp.sum(jnp.where(rid == 0, x, 0.0))
    sinv = jnp.sum(jnp.where(rid == 1, x, 0.0))
    slog = jnp.sum(jnp.where(rid == 2, x, 0.0))
    res = jnp.sqrt(ssq) * (sinv * _INV_PD) + slog * _INV_PD
    o_ref[...] = jnp.reshape(res, (1, 1))

  return pl.pallas_call(
      body,
      out_shape=jax.ShapeDtypeStruct((1, 1), jnp.float32),
  )(partials)


def kernel(mu, sigma, indices_tuple):
  partials = _sc_partials(mu, sigma, indices_tuple)
  return _tc_finish(partials)[0, 0]


# speedup vs baseline: 1.9091x; 1.0012x over previous
import functools

import jax
import jax.numpy as jnp
from jax import lax
from jax.experimental import pallas as pl
from jax.experimental.pallas import tpu as pltpu
from jax.experimental.pallas import tpu_sc as plsc

N = 16384
D = 128
P = 16384

NC = 2
NS = 16
L = 16
NW = NC * NS
PPW = P // NW
C = 64
NCH = PPW // C
R = 2 * C

_LN2 = 0.6931471805599453
_INV_PD = 1.0 / (P * D)
_EPL = PPW * D // L


def _sc_partials(mu, sigma, indices_tuple):
  mesh = plsc.VectorSubcoreMesh(core_axis_name="c", subcore_axis_name="s")

  @functools.partial(
      pl.kernel,
      out_type=jax.ShapeDtypeStruct((NW * 4, L), jnp.float32),
      mesh=mesh,
      scratch_types=[
          pltpu.VMEM((2, PPW), jnp.int32),
          pltpu.VMEM((2, R, D), jnp.float32),
          pltpu.VMEM((2, R, D), jnp.float32),
          pltpu.VMEM((4, L), jnp.float32),
          pltpu.SemaphoreType.DMA,
          pltpu.SemaphoreType.DMA,
          pltpu.SemaphoreType.DMA,
          pltpu.SemaphoreType.DMA,
      ],
  )
  def sc_kernel(mu_hbm, sg_hbm, idx_hbm, out_hbm,
                idx_v, mub, sgb, outv, m0, m1, s0, s1):
    w = lax.axis_index("s") * NC + lax.axis_index("c")
    pltpu.sync_copy(idx_hbm.at[0, pl.ds(w * PPW, PPW)], idx_v.at[0])
    pltpu.sync_copy(idx_hbm.at[1, pl.ds(w * PPW, PPW)], idx_v.at[1])
    msems = (m0, m1)
    ssems = (s0, s1)

    def fire(c):
      b = c % 2
      sl = pl.ds(c * C, C)
      cps = (
          pltpu.async_copy(mu_hbm.at[idx_v.at[0, sl]],
                           mub.at[b, pl.ds(0, C)], msems[b]),
          pltpu.async_copy(mu_hbm.at[idx_v.at[1, sl]],
                           mub.at[b, pl.ds(C, C)], msems[b]),
          pltpu.async_copy(sg_hbm.at[idx_v.at[0, sl]],
                           sgb.at[b, pl.ds(0, C)], ssems[b]),
          pltpu.async_copy(sg_hbm.at[idx_v.at[1, sl]],
                           sgb.at[b, pl.ds(C, C)], ssems[b]),
      )
      return cps

    def chunk_body(b, accs):
      def body(i, accs):
        accs = list(accs)
        for j in range(D // L):
          a_sq, a_inv, a_lm, a_e = accs[4 * (j % 2):4 * (j % 2) + 4]
          sl = pl.ds(j * L, L)
          d = mub[b, i, sl] - mub[b, C + i, sl]
          a_sq = a_sq + d * d
          sa = sgb[b, i, sl]
          sp = sgb[b, C + i, sl]
          s = sa * sa + sp * sp
          bits = lax.bitcast_convert_type(s, jnp.int32)
          a_e = a_e + lax.shift_right_logical(bits, 23)
          m = lax.bitcast_convert_type(
              (bits & jnp.int32(0x007FFFFF)) | jnp.int32(0x3F800000),
              jnp.float32)
          a_inv = a_inv + 1.0 / s
          r = (m - 1.0) / (m + 1.0)
          a_lm = a_lm + r * (2.0 + (r * r) * 0.6666666666666667)
          accs[4 * (j % 2):4 * (j % 2) + 4] = [a_sq, a_inv, a_lm, a_e]
        return tuple(accs)
      return plsc.parallel_loop(0, C, unroll=4, carry=accs)(body)

    cps = fire(0)
    zero = jnp.zeros((L,), jnp.float32)
    izero = jnp.zeros((L,), jnp.int32)
    accs = (zero, zero, zero, izero, zero, zero, zero, izero)
    for c in range(NCH):
      for cp in cps:
        cp.wait()
      if c + 1 < NCH:
        cps = fire(c + 1)
      accs = chunk_body(c % 2, accs)

    a_e = accs[3] + accs[7]
    outv[0, :] = accs[0] + accs[4]
    outv[1, :] = accs[1] + accs[5]
    outv[2, :] = (accs[2] + accs[6]
                  + (a_e.astype(jnp.float32) - 127.0 * _EPL) * _LN2)
    outv[3, :] = zero
    pltpu.sync_copy(outv, out_hbm.at[pl.ds(w * 4, 4)])

  return sc_kernel(mu, sigma, indices_tuple)


def _tc_finish(partials):
  def body(p_ref, o_ref):
    x = p_ref[...]
    rid = lax.broadcasted_iota(jnp.int32, (NW * 4, L), 0) % 4
    ssq = jnp.sum(jnp.where(rid == 0, x, 0.0))
    sinv = jnp.sum(jnp.where(rid == 1, x, 0.0))
    slog = jnp.sum(jnp.where(rid == 2, x, 0.0))
    res = jnp.sqrt(ssq) * (sinv * _INV_PD) + slog * _INV_PD
    o_ref[...] = jnp.reshape(res, (1, 1))

  return pl.pallas_call(
      body,
      out_shape=jax.ShapeDtypeStruct((1, 1), jnp.float32),
  )(partials)


def kernel(mu, sigma, indices_tuple):
  partials = _sc_partials(mu, sigma, indices_tuple)
  return _tc_finish(partials)[0, 0]
